# baseline (speedup 1.0000x reference)
import functools
import math

import jax
import jax.numpy as jnp
from jax import lax
from jax.experimental import pallas as pl
from jax.experimental.pallas import tpu as pltpu
from jax.experimental.pallas import tpu_sc as plsc

_D = 64
_SCALE = math.sqrt(_D)
_IW = 128
_C = 1024
_G = _C // _IW


def _emb_body(idx_hbm, table_hbm, out_hbm, idx_v, rows_v, sem,
              *, nc, b_per_w, n_chunks):
    wid = lax.axis_index("s") * nc + lax.axis_index("c")
    base = wid * b_per_w
    irows = b_per_w // _IW

    pltpu.sync_copy(idx_hbm.at[pl.ds(wid * irows, irows)], idx_v)

    @pl.loop(0, n_chunks)
    def _chunk(g):
        copies = [
            pltpu.async_copy(
                table_hbm.at[idx_v.at[g * _G + j]],
                rows_v.at[pl.ds(j * _IW, _IW)],
                sem,
            )
            for j in range(_G)
        ]
        for c in copies:
            c.wait()

        @pl.loop(0, _C)
        def _scale(r):
            for j in range(_D // 16):
                s = pl.ds(j * 16, 16)
                rows_v[r, s] = rows_v[r, s] * _SCALE

        pltpu.sync_copy(rows_v, out_hbm.at[pl.ds(base + g * _C, _C)])


def kernel(x, table):
    b0, b1 = x.shape
    b_total = b0 * b1
    idx = x.reshape(b_total // _IW, _IW).astype(jnp.int32)

    info = plsc.get_sparse_core_info()
    nc, ns = info.num_cores, info.num_subcores
    nw = nc * ns
    b_per_w = b_total // nw
    n_chunks = b_per_w // _C

    mesh = plsc.VectorSubcoreMesh(core_axis_name="c", subcore_axis_name="s")
    emb = pl.kernel(
        functools.partial(_emb_body, nc=nc, b_per_w=b_per_w,
                          n_chunks=n_chunks),
        out_type=jax.ShapeDtypeStruct((b_total, _D), jnp.float32),
        mesh=mesh,
        compiler_params=pltpu.CompilerParams(use_tc_tiling_on_sc=False),
        scratch_types=[
            pltpu.VMEM((b_per_w // _IW, _IW), jnp.int32),
            pltpu.VMEM((_C, _D), jnp.float32),
            pltpu.SemaphoreType.DMA,
        ],
    )
    out = emb(idx, table)
    return out.reshape(b0, b1, _D)
